# baseline (speedup 1.0000x reference)
import jax
import jax.numpy as jnp
from jax import lax
from jax.experimental import pallas as pl
from jax.experimental.pallas import tpu as pltpu
from jax.experimental.pallas import tpu_sc as plsc

_B, _L, _D = 1024, 200, 64
_CH = 128
_info = plsc.get_sparse_core_info()
_NC = _info.num_cores
_NW = _info.num_cores * _info.num_subcores
_ROWS_W = _B * _L // _NW
_NCH = _ROWS_W // _CH


def _body(seq_hbm, seg_hbm, tok_hbm, segtab_hbm, pe_hbm, out_hbm,
          idx_v, sidx_v, pe_v, seg_rows, tok_buf, sem_t, sem_s):
    c = lax.axis_index("c")
    s = lax.axis_index("s")
    wid = s * _NC + c
    r0 = wid * _NCH

    pltpu.sync_copy(seq_hbm.at[wid], idx_v)
    pltpu.sync_copy(seg_hbm.at[wid], sidx_v)
    pltpu.sync_copy(pe_hbm, pe_v)

    def chunk(ci, carry):
        tok_cp = pltpu.async_copy(tok_hbm.at[idx_v.at[ci]], tok_buf, sem_t)
        seg_cp = pltpu.async_copy(segtab_hbm.at[sidx_v.at[ci]], seg_rows, sem_s)
        tok_cp.wait()
        seg_cp.wait()
        pbase = lax.rem(ci * _CH, _L)

        def row(l, cr):
            p = lax.rem(pbase + l, _L)
            for q in range(_D // 16):
                sl = pl.ds(q * 16, 16)
                tok_buf[l, sl] = (tok_buf[l, sl] + pe_v[p, sl]
                                  + seg_rows[l, sl])
            return cr

        lax.fori_loop(0, _CH, row, 0)
        pltpu.sync_copy(tok_buf, out_hbm.at[pl.ds((r0 + ci) * _CH, _CH)])
        return carry

    lax.fori_loop(0, _NCH, chunk, 0)


def kernel(sequence, segment_label, token_table, seg_table, pe):
    b, l = sequence.shape
    v, d = token_table.shape
    seqf = sequence.astype(jnp.int32).reshape(_NW, _NCH, _CH)
    segf = segment_label.astype(jnp.int32).reshape(_NW, _NCH, _CH)
    pe2 = pe[0, :l, :]

    k = pl.kernel(
        _body,
        out_type=jax.ShapeDtypeStruct((b * l, d), jnp.float32),
        mesh=plsc.VectorSubcoreMesh(core_axis_name="c", subcore_axis_name="s"),
        compiler_params=pltpu.CompilerParams(use_tc_tiling_on_sc=False),
        scratch_types=[
            pltpu.VMEM((_NCH, _CH), jnp.int32),
            pltpu.VMEM((_NCH, _CH), jnp.int32),
            pltpu.VMEM((_L, _D), jnp.float32),
            pltpu.VMEM((_CH, _D), jnp.float32),
            pltpu.VMEM((_CH, _D), jnp.float32),
            pltpu.SemaphoreType.DMA,
            pltpu.SemaphoreType.DMA,
        ],
    )
    out = k(seqf, segf, token_table, seg_table, pe2)
    return out.reshape(b, l, d)
